# baseline (speedup 1.0000x reference)
import functools

import jax
import jax.numpy as jnp
from jax import lax
from jax.experimental import pallas as pl
from jax.experimental.pallas import tpu as pltpu
from jax.experimental.pallas import tpu_sc as plsc

B = 16384
K = 128
NC = 2
NS = 16
L = 16
NW = NC * NS
BPW = B // NW
CH = 128
NCH = BPW // CH
GPC = CH // L

_mesh = plsc.VectorSubcoreMesh(core_axis_name="c", subcore_axis_name="s")


@functools.partial(
    pl.kernel,
    out_type=jax.ShapeDtypeStruct((B,), jnp.float32),
    mesh=_mesh,
    compiler_params=pltpu.CompilerParams(needs_layout_passes=False),
    scratch_types=[
        pltpu.VMEM((BPW,), jnp.int32),
        pltpu.VMEM((BPW,), jnp.int32),
        pltpu.VMEM((NCH, CH, K), jnp.float32),
        pltpu.VMEM((2, CH, K), jnp.float32),
        pltpu.VMEM((GPC * L * L,), jnp.float32),
        pltpu.VMEM((BPW,), jnp.float32),
        pltpu.SemaphoreType.DMA,
        pltpu.SemaphoreType.DMA,
        pltpu.SemaphoreType.DMA,
    ],
)
def _mf_kernel(stock_hbm, field_hbm, sw_hbm, fw_hbm, out_hbm,
               sidx, fidx, srows, frows, colbuf, outv, sem0, sem1, sem2):
    wid = lax.axis_index("s") * NC + lax.axis_index("c")
    base = wid * BPW

    sems = (sem0, sem1)
    iota = lax.iota(jnp.int32, L)

    di1 = pltpu.async_copy(stock_hbm.at[pl.ds(base, BPW)], sidx, sem0)
    di2 = pltpu.async_copy(field_hbm.at[pl.ds(base, BPW)], fidx, sem1)
    di1.wait()
    di2.wait()

    for c in range(NCH):
        pltpu.async_copy(sw_hbm.at[sidx.at[pl.ds(c * CH, CH)]],
                         srows.at[c], sems[0])

    def start_field(c, buf):
        pltpu.async_copy(fw_hbm.at[fidx.at[pl.ds(c * CH, CH)]],
                         frows.at[buf], sems[1])

    def wait_chunk(buf):
        pltpu.make_async_copy(sw_hbm.at[pl.ds(0, CH)], srows.at[0],
                              sems[0]).wait()
        pltpu.make_async_copy(fw_hbm.at[pl.ds(0, CH)], frows.at[buf],
                              sems[1]).wait()

    def compute_chunk(c, buf):
        sb = srows.at[c]
        fb = frows.at[buf]

        @plsc.parallel_loop(0, GPC, 1, unroll=1)
        def gbody(g):
            gb = g * L
            cb = g * (L * L)
            for j in range(L):
                bj = gb + j
                acc0 = sb[bj, pl.ds(0, L)] * fb[bj, pl.ds(0, L)]
                acc1 = sb[bj, pl.ds(L, L)] * fb[bj, pl.ds(L, L)]
                for k in range(2, K // L, 2):
                    acc0 = acc0 + (sb[bj, pl.ds(k * L, L)]
                                   * fb[bj, pl.ds(k * L, L)])
                    acc1 = acc1 + (sb[bj, pl.ds((k + 1) * L, L)]
                                   * fb[bj, pl.ds((k + 1) * L, L)])
                colbuf[pl.ds(cb + j * L, L)] = acc0 + acc1
            col = cb + iota * L
            tot0 = plsc.load_gather(colbuf, [col])
            tot1 = plsc.load_gather(colbuf, [col + 1])
            for i in range(2, L, 2):
                tot0 = tot0 + plsc.load_gather(colbuf, [col + i])
                tot1 = tot1 + plsc.load_gather(colbuf, [col + i + 1])
            tot = tot0 + tot1
            outv[pl.ds(c * CH + gb, L)] = tot

    start_field(0, 0)

    def chunk_body(c, carry):
        s = lax.rem(c, 2)
        wait_chunk(s)

        @pl.when(c + 1 < NCH)
        def _prefetch():
            start_field(c + 1, 1 - s)

        compute_chunk(c, s)
        pltpu.async_copy(outv.at[pl.ds(c * CH, CH)],
                         out_hbm.at[pl.ds(base + c * CH, CH)], sem2)
        return carry

    lax.fori_loop(0, NCH, chunk_body, 0)

    for c in range(NCH):
        pltpu.make_async_copy(outv.at[pl.ds(0, CH)],
                              out_hbm.at[pl.ds(base, CH)], sem2).wait()


def kernel(stock, field, stock_intr_weight, field_corr_weight):
    return _mf_kernel(stock.astype(jnp.int32), field.astype(jnp.int32),
                      stock_intr_weight, field_corr_weight)


# speedup vs baseline: 1.0058x; 1.0058x over previous
import functools

import jax
import jax.numpy as jnp
from jax import lax
from jax.experimental import pallas as pl
from jax.experimental.pallas import tpu as pltpu
from jax.experimental.pallas import tpu_sc as plsc

B = 16384
K = 128
NC = 2
NS = 16
L = 16
NW = NC * NS
BPW = B // NW
CH = 128
NCH = BPW // CH
GPC = CH // L

_mesh = plsc.VectorSubcoreMesh(core_axis_name="c", subcore_axis_name="s")


@functools.partial(
    pl.kernel,
    out_type=jax.ShapeDtypeStruct((B,), jnp.float32),
    mesh=_mesh,
    compiler_params=pltpu.CompilerParams(needs_layout_passes=False),
    scratch_types=[
        pltpu.VMEM((BPW,), jnp.int32),
        pltpu.VMEM((BPW,), jnp.int32),
        pltpu.VMEM((NCH, CH, K), jnp.float32),
        pltpu.VMEM((2, CH, K), jnp.float32),
        pltpu.VMEM((GPC * L * L,), jnp.float32),
        pltpu.VMEM((BPW,), jnp.float32),
        pltpu.SemaphoreType.DMA,
        pltpu.SemaphoreType.DMA,
    ],
)
def _mf_kernel(stock_hbm, field_hbm, sw_hbm, fw_hbm, out_hbm,
               sidx, fidx, srows, frows, colbuf, outv, sem0, sem1):
    wid = lax.axis_index("s") * NC + lax.axis_index("c")
    base = wid * BPW

    sems = (sem0, sem1)
    iota = lax.iota(jnp.int32, L)

    di1 = pltpu.async_copy(stock_hbm.at[pl.ds(base, BPW)], sidx, sem0)
    di2 = pltpu.async_copy(field_hbm.at[pl.ds(base, BPW)], fidx, sem1)
    di1.wait()
    di2.wait()

    for c in range(NCH):
        pltpu.async_copy(sw_hbm.at[sidx.at[pl.ds(c * CH, CH)]],
                         srows.at[c], sems[0])

    def start_field(c, buf):
        pltpu.async_copy(fw_hbm.at[fidx.at[pl.ds(c * CH, CH)]],
                         frows.at[buf], sems[1])

    def wait_chunk(buf):
        pltpu.make_async_copy(sw_hbm.at[pl.ds(0, CH)], srows.at[0],
                              sems[0]).wait()
        pltpu.make_async_copy(fw_hbm.at[pl.ds(0, CH)], frows.at[buf],
                              sems[1]).wait()

    def compute_chunk(c, buf):
        sb = srows.at[c]
        fb = frows.at[buf]

        @plsc.parallel_loop(0, GPC, 1, unroll=1)
        def gbody(g):
            gb = g * L
            cb = g * (L * L)
            for j in range(L):
                bj = gb + j
                acc0 = sb[bj, pl.ds(0, L)] * fb[bj, pl.ds(0, L)]
                acc1 = sb[bj, pl.ds(L, L)] * fb[bj, pl.ds(L, L)]
                for k in range(2, K // L, 2):
                    acc0 = acc0 + (sb[bj, pl.ds(k * L, L)]
                                   * fb[bj, pl.ds(k * L, L)])
                    acc1 = acc1 + (sb[bj, pl.ds((k + 1) * L, L)]
                                   * fb[bj, pl.ds((k + 1) * L, L)])
                colbuf[pl.ds(cb + j * L, L)] = acc0 + acc1
            col = cb + iota * L
            tot0 = plsc.load_gather(colbuf, [col])
            tot1 = plsc.load_gather(colbuf, [col + 1])
            for i in range(2, L, 2):
                tot0 = tot0 + plsc.load_gather(colbuf, [col + i])
                tot1 = tot1 + plsc.load_gather(colbuf, [col + i + 1])
            tot = tot0 + tot1
            outv[pl.ds(c * CH + gb, L)] = tot

    start_field(0, 0)

    def chunk_body(c, carry):
        s = lax.rem(c, 2)
        wait_chunk(s)

        @pl.when(c + 1 < NCH)
        def _prefetch():
            start_field(c + 1, 1 - s)

        compute_chunk(c, s)
        return carry

    lax.fori_loop(0, NCH, chunk_body, 0)

    pltpu.sync_copy(outv, out_hbm.at[pl.ds(base, BPW)])


def kernel(stock, field, stock_intr_weight, field_corr_weight):
    return _mf_kernel(stock.astype(jnp.int32), field.astype(jnp.int32),
                      stock_intr_weight, field_corr_weight)
